# baseline (speedup 1.0000x reference)
import functools

import jax
import jax.numpy as jnp
from jax import lax
from jax.experimental import pallas as pl
from jax.experimental.pallas import tpu as pltpu
from jax.experimental.pallas import tpu_sc as plsc

_NUM_ROWS = 100
_DIM = 3
_PAD = 8
_BATCH = 16384

_NW = 16
_BPW = _BATCH // _NW

_MESH = plsc.VectorSubcoreMesh(
    core_axis_name="c", subcore_axis_name="s", num_cores=1
)


@functools.partial(
    pl.kernel,
    mesh=_MESH,
    out_type=jax.ShapeDtypeStruct((_BATCH, _PAD), jnp.float32),
    scratch_types=[
        pltpu.VMEM((_BPW,), jnp.int32),
        pltpu.VMEM((_BPW, _PAD), jnp.float32),
        pltpu.VMEM((_NUM_ROWS, _PAD), jnp.float32),
        pltpu.VMEM_SHARED((_NUM_ROWS, _PAD), jnp.float32),
        pltpu.SemaphoreType.DMA,
    ],
    compiler_params=pltpu.CompilerParams(use_tc_tiling_on_sc=False),
)
def _gather_sc(ids_hbm, table_hbm, out_hbm, idx_v, rows_v, tab_v, tab_sh, sem):
    wid = lax.axis_index("s")
    base = wid * _BPW
    pltpu.sync_copy(table_hbm, tab_v.at[:, pl.ds(0, _DIM)])
    pltpu.sync_copy(ids_hbm.at[pl.ds(base, _BPW)], idx_v)

    @pl.when(wid == 0)
    def _():
        pltpu.sync_copy(tab_v, tab_sh)

    plsc.subcore_barrier()
    pltpu.async_copy(tab_sh.at[idx_v], rows_v, sem).wait()
    pltpu.sync_copy(rows_v, out_hbm.at[pl.ds(base, _BPW)])


def kernel(entity_ids, entity_table):
    ids = entity_ids.astype(jnp.int32)
    return _gather_sc(ids, entity_table)[:, :_DIM]


# speedup vs baseline: 1.0678x; 1.0678x over previous
import functools

import jax
import jax.numpy as jnp
from jax import lax
from jax.experimental import pallas as pl
from jax.experimental.pallas import tpu as pltpu
from jax.experimental.pallas import tpu_sc as plsc

_NUM_ROWS = 100
_DIM = 3
_PAD = 16
_BATCH = 16384

_NW = 16
_BPW = _BATCH // _NW
_H = _BPW // 2

_MESH = plsc.VectorSubcoreMesh(
    core_axis_name="c", subcore_axis_name="s", num_cores=1
)


@functools.partial(
    pl.kernel,
    mesh=_MESH,
    out_type=jax.ShapeDtypeStruct((_BATCH, _PAD), jnp.float32),
    scratch_types=[
        pltpu.VMEM((_BPW,), jnp.int32),
        pltpu.VMEM((_BPW, _PAD), jnp.float32),
        pltpu.VMEM((_NUM_ROWS, _PAD), jnp.float32),
        pltpu.VMEM_SHARED((_NUM_ROWS, _PAD), jnp.float32),
        pltpu.SemaphoreType.DMA,
        pltpu.SemaphoreType.DMA,
        pltpu.SemaphoreType.DMA,
    ],
    compiler_params=pltpu.CompilerParams(use_tc_tiling_on_sc=False),
)
def _gather_sc(
    ids_hbm, table_hbm, out_hbm, idx_v, rows_v, tab_v, tab_sh, sem, semi, semw
):
    wid = lax.axis_index("s")
    base = wid * _BPW
    idx_cp = pltpu.async_copy(ids_hbm.at[pl.ds(base, _BPW)], idx_v, semi)

    @pl.when(wid == 0)
    def _():
        pltpu.sync_copy(table_hbm, tab_v.at[:, pl.ds(0, _DIM)])
        pltpu.sync_copy(tab_v, tab_sh)

    idx_cp.wait()
    plsc.subcore_barrier()
    pltpu.async_copy(
        tab_sh.at[idx_v.at[pl.ds(0, _H)]], rows_v.at[pl.ds(0, _H)], sem
    ).wait()
    w0 = pltpu.async_copy(
        rows_v.at[pl.ds(0, _H)], out_hbm.at[pl.ds(base, _H)], semw
    )
    pltpu.async_copy(
        tab_sh.at[idx_v.at[pl.ds(_H, _H)]], rows_v.at[pl.ds(_H, _H)], sem
    ).wait()
    w1 = pltpu.async_copy(
        rows_v.at[pl.ds(_H, _H)], out_hbm.at[pl.ds(base + _H, _H)], semw
    )
    w0.wait()
    w1.wait()


def kernel(entity_ids, entity_table):
    ids = entity_ids.astype(jnp.int32)
    return _gather_sc(ids, entity_table)[:, :_DIM]
